# baseline (speedup 1.0000x reference)
import functools

import jax
import jax.numpy as jnp
from jax import lax
from jax.experimental import pallas as pl
from jax.experimental.pallas import tpu as pltpu

_BIN_SIZE = 4096


def _ffd_bins(lengths, bin_size):
    order = sorted(range(len(lengths)), key=lambda i: lengths[i], reverse=True)
    bins = [[]]
    for idx in order:
        L = lengths[idx]
        placed = False
        for b in bins:
            if sum(lengths[j] for j in b) + L <= bin_size:
                b.append(idx)
                placed = True
                break
        if not placed:
            bins.append([idx])
    return bins


def kernel(seq0, seq1, seq2, seq3, seq4, seq5, seq6, seq7):
    seqs = [seq0, seq1, seq2, seq3, seq4, seq5, seq6, seq7]
    lengths = [int(s.shape[0]) for s in seqs]
    hidden = int(seqs[0].shape[1])
    bins = _ffd_bins(lengths, _BIN_SIZE)
    used = [sum(lengths[j] for j in b) for b in bins]
    max_len = max(used)
    nbins = len(bins)

    copies = []
    pads = []
    for b, members in enumerate(bins):
        off = 0
        for j in members:
            copies.append((b, off, j))
            off += lengths[j]
        if off < max_len:
            pads.append((b, off, max_len - off))

    ZROWS = 512
    ndma = len(copies) + sum((p + ZROWS - 1) // ZROWS for (_, _, p) in pads) + 1

    def body(*refs):
        seq_refs = refs[:8]
        out_ref, mask_ref, zbuf, mbuf, sems = refs[8:]
        handles = []
        i = 0
        for (b, dst0, j) in copies:
            h = pltpu.make_async_copy(
                seq_refs[j], out_ref.at[b, pl.ds(dst0, lengths[j]), :],
                sems.at[i])
            h.start()
            handles.append(h)
            i += 1

        zbuf[...] = jnp.zeros((ZROWS, hidden), jnp.float32)
        col = lax.broadcasted_iota(jnp.int32, (1, max_len), 1)
        for b in range(nbins):
            mbuf[pl.ds(b, 1), :] = jnp.where(
                col < used[b], jnp.float32(1.0), jnp.float32(0.0))

        for (b, off, p) in pads:
            for z0 in range(0, p, ZROWS):
                zc = min(ZROWS, p - z0)
                h = pltpu.make_async_copy(
                    zbuf.at[pl.ds(0, zc)],
                    out_ref.at[b, pl.ds(off + z0, zc), :],
                    sems.at[i])
                h.start()
                handles.append(h)
                i += 1

        h = pltpu.make_async_copy(mbuf, mask_ref, sems.at[i])
        h.start()
        handles.append(h)

        for h in handles:
            h.wait()

    return pl.pallas_call(
        body,
        in_specs=[pl.BlockSpec(memory_space=pltpu.MemorySpace.HBM)] * 8,
        out_specs=(
            pl.BlockSpec(memory_space=pltpu.MemorySpace.HBM),
            pl.BlockSpec(memory_space=pltpu.MemorySpace.HBM),
        ),
        out_shape=(
            jax.ShapeDtypeStruct((nbins, max_len, hidden), jnp.float32),
            jax.ShapeDtypeStruct((nbins, max_len), jnp.float32),
        ),
        scratch_shapes=[
            pltpu.VMEM((ZROWS, hidden), jnp.float32),
            pltpu.VMEM((nbins, max_len), jnp.float32),
            pltpu.SemaphoreType.DMA((ndma,)),
        ],
    )(*seqs)


# speedup vs baseline: 31.0368x; 31.0368x over previous
import functools

import jax
import jax.numpy as jnp
from jax import lax
from jax.experimental import pallas as pl
from jax.experimental.pallas import tpu as pltpu
from jax.experimental.pallas import tpu_sc as plsc

_BIN_SIZE = 4096
_CHUNK = 16
_NBUF = 4
_AHEAD = 4


def _ffd_bins(lengths, bin_size):
    order = sorted(range(len(lengths)), key=lambda i: lengths[i], reverse=True)
    bins = [[]]
    for idx in order:
        L = lengths[idx]
        placed = False
        for b in bins:
            if sum(lengths[j] for j in b) + L <= bin_size:
                b.append(idx)
                placed = True
                break
        if not placed:
            bins.append([idx])
    return bins


def kernel(seq0, seq1, seq2, seq3, seq4, seq5, seq6, seq7):
    seqs = [seq0, seq1, seq2, seq3, seq4, seq5, seq6, seq7]
    lengths = [int(s.shape[0]) for s in seqs]
    hidden = int(seqs[0].shape[1])
    bins = _ffd_bins(lengths, _BIN_SIZE)
    used = [sum(lengths[j] for j in b) for b in bins]
    max_len = max(used)
    nbins = len(bins)

    copies = []
    pads = []
    for b, members in enumerate(bins):
        off = 0
        for j in members:
            copies.append((b, off, j))
            off += lengths[j]
        if off < max_len:
            pads.append((b, off, max_len - off))

    info = plsc.get_sparse_core_info()
    NC, NS = int(info.num_cores), int(info.num_subcores)
    W = NC * NS

    assert all(L % W == 0 for L in lengths), lengths
    assert all(p % W == 0 and (p // W) % 16 == 0 for (_, _, p) in pads)
    assert max_len % (W * 16) == 0 and hidden % 16 == 0

    km = max_len // W

    plan = []
    for (b, dst0, j) in copies:
        share = lengths[j] // W
        for rel in range(0, share, _CHUNK):
            plan.append((j, b, dst0, share, rel, min(_CHUNK, share - rel)))
    nchunks = len(plan)

    _SPBUF = 2
    path = [1 if (i % 3 == 2) else 0 for i in range(nchunks)]
    slot = [0] * nchunks
    prev_user = [None] * nchunks
    counters = [0, 0]
    last_on_slot = {}
    for i in range(nchunks):
        p = path[i]
        nb = _SPBUF if p == 1 else _NBUF
        slot[i] = counters[p] % nb
        counters[p] += 1
        key = (p, slot[i])
        prev_user[i] = last_on_slot.get(key)
        last_on_slot[key] = i

    mesh = plsc.VectorSubcoreMesh(core_axis_name="c", subcore_axis_name="s")

    @functools.partial(
        pl.kernel,
        mesh=mesh,
        out_type=(
            jax.ShapeDtypeStruct((nbins, max_len, hidden), jnp.float32),
            jax.ShapeDtypeStruct((nbins, max_len), jnp.float32),
        ),
        scratch_types=(
            [pltpu.VMEM((_CHUNK, hidden), jnp.float32)] * _NBUF
            + [pltpu.VMEM_SHARED((NS, _SPBUF, _CHUNK, hidden), jnp.float32)]
            + [
                pltpu.VMEM((16, hidden), jnp.float32),
                pltpu.VMEM((nbins, km), jnp.float32),
            ]
            + [pltpu.SemaphoreType.DMA] * (2 * (_NBUF + _SPBUF))
            + [pltpu.SemaphoreType.DMA]
        ),
    )
    def _pack(s0, s1, s2, s3, s4, s5, s6, s7, out_ref, mask_ref, *scratch):
        seq_refs = [s0, s1, s2, s3, s4, s5, s6, s7]
        bufs = list(scratch[:_NBUF])
        shared = scratch[_NBUF]
        zbuf, mbuf = scratch[_NBUF + 1], scratch[_NBUF + 2]
        nsem = _NBUF + _SPBUF
        rsems = list(scratch[_NBUF + 3:_NBUF + 3 + nsem])
        wsems = list(scratch[_NBUF + 3 + nsem:_NBUF + 3 + 2 * nsem])
        zsem = scratch[_NBUF + 3 + 2 * nsem]
        sid = lax.axis_index("s")
        wid = sid * NC + lax.axis_index("c")

        rh = [None] * nchunks
        wh = [None] * nchunks

        def buf_at(i, cnt):
            if path[i] == 0:
                return bufs[slot[i]].at[pl.ds(0, cnt)]
            return shared.at[sid, slot[i], pl.ds(0, cnt), :]

        def sem_idx(i):
            return slot[i] if path[i] == 0 else _NBUF + slot[i]

        def start_read(i):
            j, b, dst0, share, rel, cnt = plan[i]
            rh[i] = pltpu.async_copy(
                seq_refs[j].at[pl.ds(wid * share + rel, cnt), :],
                buf_at(i, cnt),
                rsems[sem_idx(i)],
            )

        def start_write(i):
            j, b, dst0, share, rel, cnt = plan[i]
            wh[i] = pltpu.async_copy(
                buf_at(i, cnt),
                out_ref.at[b, pl.ds(dst0 + wid * share + rel, cnt), :],
                wsems[sem_idx(i)],
            )

        for i in range(min(_AHEAD, nchunks)):
            start_read(i)

        zv = jnp.zeros((16,), jnp.float32)
        def _zrow(i, c):
            def _zcol(jj, cc):
                for u in range(4):
                    zbuf[i, pl.ds(jj * 64 + u * 16, 16)] = zv
                return cc
            return lax.fori_loop(0, hidden // 64, _zcol, c)
        lax.fori_loop(0, 16, _zrow, 0)

        aux = []
        for (b, off, p) in pads:
            kp = p // W
            base = off + wid * kp
            for c0 in range(0, kp, 16):
                aux.append(
                    pltpu.async_copy(
                        zbuf, out_ref.at[b, pl.ds(base + c0, 16), :], zsem
                    )
                )

        iot = lax.iota(jnp.int32, 16)
        col0 = wid * km
        for b in range(nbins):
            for jj in range(km // 16):
                col = col0 + jj * 16 + iot
                mbuf[b, pl.ds(jj * 16, 16)] = jnp.where(
                    col < used[b], jnp.float32(1.0), jnp.float32(0.0)
                )
            aux.append(
                pltpu.async_copy(
                    mbuf.at[pl.ds(b, 1)],
                    mask_ref.at[pl.ds(b, 1), pl.ds(col0, km)],
                    zsem,
                )
            )

        waited = set()
        for i in range(nchunks):
            if i + _AHEAD < nchunks:
                prev = prev_user[i + _AHEAD]
                if prev is not None:
                    wh[prev].wait()
                    waited.add(prev)
                start_read(i + _AHEAD)
            rh[i].wait()
            start_write(i)

        for i in range(nchunks):
            if i not in waited:
                wh[i].wait()
        for h in aux:
            h.wait()

    return _pack(*seqs)


# speedup vs baseline: 44.5336x; 1.4349x over previous
import functools

import jax
import jax.numpy as jnp
from jax import lax
from jax.experimental import pallas as pl
from jax.experimental.pallas import tpu as pltpu

_BIN_SIZE = 4096
_CHUNK = 256
_NBUF = 8
_AHEAD = 4


def _ffd_bins(lengths, bin_size):
    order = sorted(range(len(lengths)), key=lambda i: lengths[i], reverse=True)
    bins = [[]]
    for idx in order:
        L = lengths[idx]
        placed = False
        for b in bins:
            if sum(lengths[j] for j in b) + L <= bin_size:
                b.append(idx)
                placed = True
                break
        if not placed:
            bins.append([idx])
    return bins


def kernel(seq0, seq1, seq2, seq3, seq4, seq5, seq6, seq7):
    seqs = [seq0, seq1, seq2, seq3, seq4, seq5, seq6, seq7]
    lengths = [int(s.shape[0]) for s in seqs]
    hidden = int(seqs[0].shape[1])
    bins = _ffd_bins(lengths, _BIN_SIZE)
    used = [sum(lengths[j] for j in b) for b in bins]
    max_len = max(used)
    nbins = len(bins)

    copies = []
    pads = []
    for b, members in enumerate(bins):
        off = 0
        for j in members:
            copies.append((b, off, j))
            off += lengths[j]
        if off < max_len:
            pads.append((b, off, max_len - off))

    plan = []
    for (b, dst0, j) in copies:
        for rel in range(0, lengths[j], _CHUNK):
            cnt = min(_CHUNK, lengths[j] - rel)
            plan.append((j, b, dst0 + rel, rel, cnt))
    nchunks = len(plan)

    nzdma = sum((p + _CHUNK - 1) // _CHUNK for (_, _, p) in pads)

    def body(*refs):
        seq_refs = refs[:8]
        out_ref, mask_ref = refs[8], refs[9]
        bufs = list(refs[10:10 + _NBUF])
        zbuf, mbuf = refs[10 + _NBUF], refs[11 + _NBUF]
        rsems = refs[12 + _NBUF]
        wsems = refs[13 + _NBUF]
        zsems = refs[14 + _NBUF]

        rh = [None] * nchunks
        wh = [None] * nchunks

        def start_read(i):
            j, b, dst, src, cnt = plan[i]
            rh[i] = pltpu.make_async_copy(
                seq_refs[j].at[pl.ds(src, cnt), :],
                bufs[i % _NBUF].at[pl.ds(0, cnt)],
                rsems.at[i % _NBUF],
            )
            rh[i].start()

        def start_write(i):
            j, b, dst, src, cnt = plan[i]
            wh[i] = pltpu.make_async_copy(
                bufs[i % _NBUF].at[pl.ds(0, cnt)],
                out_ref.at[b, pl.ds(dst, cnt), :],
                wsems.at[i % _NBUF],
            )
            wh[i].start()

        for i in range(min(_AHEAD, nchunks)):
            start_read(i)

        zbuf[...] = jnp.zeros((_CHUNK, hidden), jnp.float32)
        col = lax.broadcasted_iota(jnp.int32, (1, max_len), 1)
        for b in range(nbins):
            mbuf[pl.ds(b, 1), :] = jnp.where(
                col < used[b], jnp.float32(1.0), jnp.float32(0.0))

        aux = []
        zi = 0
        for (b, off, p) in pads:
            for z0 in range(0, p, _CHUNK):
                zc = min(_CHUNK, p - z0)
                h = pltpu.make_async_copy(
                    zbuf.at[pl.ds(0, zc)],
                    out_ref.at[b, pl.ds(off + z0, zc), :],
                    zsems.at[zi],
                )
                h.start()
                aux.append(h)
                zi += 1
        h = pltpu.make_async_copy(mbuf, mask_ref, zsems.at[zi])
        h.start()
        aux.append(h)

        waited = set()
        for i in range(nchunks):
            if i + _AHEAD < nchunks:
                prev = i + _AHEAD - _NBUF
                if prev >= 0:
                    wh[prev].wait()
                    waited.add(prev)
                start_read(i + _AHEAD)
            rh[i].wait()
            start_write(i)

        for i in range(nchunks):
            if i not in waited:
                wh[i].wait()
        for h in aux:
            h.wait()

    return pl.pallas_call(
        body,
        in_specs=[pl.BlockSpec(memory_space=pltpu.MemorySpace.HBM)] * 8,
        out_specs=(
            pl.BlockSpec(memory_space=pltpu.MemorySpace.HBM),
            pl.BlockSpec(memory_space=pltpu.MemorySpace.HBM),
        ),
        out_shape=(
            jax.ShapeDtypeStruct((nbins, max_len, hidden), jnp.float32),
            jax.ShapeDtypeStruct((nbins, max_len), jnp.float32),
        ),
        scratch_shapes=(
            [pltpu.VMEM((_CHUNK, hidden), jnp.float32)] * _NBUF
            + [
                pltpu.VMEM((_CHUNK, hidden), jnp.float32),
                pltpu.VMEM((nbins, max_len), jnp.float32),
                pltpu.SemaphoreType.DMA((_NBUF,)),
                pltpu.SemaphoreType.DMA((_NBUF,)),
                pltpu.SemaphoreType.DMA((nzdma + 1,)),
            ]
        ),
    )(*seqs)


# speedup vs baseline: 45.6086x; 1.0241x over previous
import functools

import jax
import jax.numpy as jnp
from jax import lax
from jax.experimental import pallas as pl
from jax.experimental.pallas import tpu as pltpu

_BIN_SIZE = 4096
_CHUNK = 512
_NBUF = 8
_AHEAD = 4


def _ffd_bins(lengths, bin_size):
    order = sorted(range(len(lengths)), key=lambda i: lengths[i], reverse=True)
    bins = [[]]
    for idx in order:
        L = lengths[idx]
        placed = False
        for b in bins:
            if sum(lengths[j] for j in b) + L <= bin_size:
                b.append(idx)
                placed = True
                break
        if not placed:
            bins.append([idx])
    return bins


def kernel(seq0, seq1, seq2, seq3, seq4, seq5, seq6, seq7):
    seqs = [seq0, seq1, seq2, seq3, seq4, seq5, seq6, seq7]
    lengths = [int(s.shape[0]) for s in seqs]
    hidden = int(seqs[0].shape[1])
    bins = _ffd_bins(lengths, _BIN_SIZE)
    used = [sum(lengths[j] for j in b) for b in bins]
    max_len = max(used)
    nbins = len(bins)

    copies = []
    pads = []
    for b, members in enumerate(bins):
        off = 0
        for j in members:
            copies.append((b, off, j))
            off += lengths[j]
        if off < max_len:
            pads.append((b, off, max_len - off))

    plan = []
    for (b, dst0, j) in copies:
        for rel in range(0, lengths[j], _CHUNK):
            cnt = min(_CHUNK, lengths[j] - rel)
            plan.append((j, b, dst0 + rel, rel, cnt))
    nchunks = len(plan)

    nzdma = sum((p + _CHUNK - 1) // _CHUNK for (_, _, p) in pads)

    def body(*refs):
        seq_refs = refs[:8]
        out_ref, mask_ref = refs[8], refs[9]
        bufs = list(refs[10:10 + _NBUF])
        zbuf, mbuf = refs[10 + _NBUF], refs[11 + _NBUF]
        rsems = refs[12 + _NBUF]
        wsems = refs[13 + _NBUF]
        zsems = refs[14 + _NBUF]

        rh = [None] * nchunks
        wh = [None] * nchunks

        def start_read(i):
            j, b, dst, src, cnt = plan[i]
            rh[i] = pltpu.make_async_copy(
                seq_refs[j].at[pl.ds(src, cnt), :],
                bufs[i % _NBUF].at[pl.ds(0, cnt)],
                rsems.at[i % _NBUF],
            )
            rh[i].start()

        def start_write(i):
            j, b, dst, src, cnt = plan[i]
            wh[i] = pltpu.make_async_copy(
                bufs[i % _NBUF].at[pl.ds(0, cnt)],
                out_ref.at[b, pl.ds(dst, cnt), :],
                wsems.at[i % _NBUF],
            )
            wh[i].start()

        for i in range(min(_AHEAD, nchunks)):
            start_read(i)

        zbuf[...] = jnp.zeros((_CHUNK, hidden), jnp.float32)
        col = lax.broadcasted_iota(jnp.int32, (1, max_len), 1)
        for b in range(nbins):
            mbuf[pl.ds(b, 1), :] = jnp.where(
                col < used[b], jnp.float32(1.0), jnp.float32(0.0))

        aux = []
        zi = 0
        for (b, off, p) in pads:
            for z0 in range(0, p, _CHUNK):
                zc = min(_CHUNK, p - z0)
                h = pltpu.make_async_copy(
                    zbuf.at[pl.ds(0, zc)],
                    out_ref.at[b, pl.ds(off + z0, zc), :],
                    zsems.at[zi],
                )
                h.start()
                aux.append(h)
                zi += 1
        h = pltpu.make_async_copy(mbuf, mask_ref, zsems.at[zi])
        h.start()
        aux.append(h)

        waited = set()
        for i in range(nchunks):
            if i + _AHEAD < nchunks:
                prev = i + _AHEAD - _NBUF
                if prev >= 0:
                    wh[prev].wait()
                    waited.add(prev)
                start_read(i + _AHEAD)
            rh[i].wait()
            start_write(i)

        for i in range(nchunks):
            if i not in waited:
                wh[i].wait()
        for h in aux:
            h.wait()

    return pl.pallas_call(
        body,
        in_specs=[pl.BlockSpec(memory_space=pltpu.MemorySpace.HBM)] * 8,
        out_specs=(
            pl.BlockSpec(memory_space=pltpu.MemorySpace.HBM),
            pl.BlockSpec(memory_space=pltpu.MemorySpace.HBM),
        ),
        out_shape=(
            jax.ShapeDtypeStruct((nbins, max_len, hidden), jnp.float32),
            jax.ShapeDtypeStruct((nbins, max_len), jnp.float32),
        ),
        scratch_shapes=(
            [pltpu.VMEM((_CHUNK, hidden), jnp.float32)] * _NBUF
            + [
                pltpu.VMEM((_CHUNK, hidden), jnp.float32),
                pltpu.VMEM((nbins, max_len), jnp.float32),
                pltpu.SemaphoreType.DMA((_NBUF,)),
                pltpu.SemaphoreType.DMA((_NBUF,)),
                pltpu.SemaphoreType.DMA((nzdma + 1,)),
            ]
        ),
    )(*seqs)
